# baseline (speedup 1.0000x reference)
import jax
import jax.numpy as jnp
from jax.experimental import pallas as pl
from jax.experimental.pallas import tpu as pltpu

CHANS = (32, 32, 64, 64, 128, 128)
B = 16
L0 = 2048
T = 32
H = 128
NE = 3
NB = NE * B
NC = 7


def _fused_kernel(xp_ref, xd_ref, xe_ref, w0_ref, b0_ref, wa_ref, wc_ref, bc_ref,
                  wih0, wih1, wih2, whh0, whh1, whh2,
                  bih0, bih1, bih2, bhh0, bhh1, bhh2,
                  e1p, e1d, e1e, e2p, e2d, e2e, re_ref, dp_ref,
                  out_ref, l_ref, gi_scr, q_scr):
    f32 = jnp.float32
    x_ref = (xp_ref, xd_ref, xe_ref)
    wih_refs = (wih0, wih1, wih2)
    whh_refs = (whh0, whh1, whh2)
    bih_refs = (bih0, bih1, bih2)
    bhh_refs = (bhh0, bhh1, bhh2)
    cdims = (((1,), (1,)), ((), ()))

    for e in range(NE):
        x = x_ref[e][...].T
        xr = x.reshape(L0 // 2, 2, B)
        ev = xr[:, 0, :]
        od = xr[:, 1, :]
        pod = jnp.concatenate(
            [jnp.zeros((1, B), f32), od[:-1]], axis=0)
        x48 = jnp.concatenate([pod, ev, od], axis=1)
        h = x48 @ w0_ref[e] + b0_ref[e]
        h = jnp.maximum(h, 0.0).reshape(L0 // 2 * B, H)

        for l in range(5):
            rows = h.shape[0]
            hr = h.reshape(rows // (2 * B), 2 * B, H)
            evf = hr[:, :B, :].reshape(rows // 2, H)
            odf = hr[:, B:, :].reshape(rows // 2, H)
            a = evf @ wa_ref[l, e] + bc_ref[l, e]
            c = (odf @ wc_ref[l, e]).reshape(rows // (2 * B), B, 2 * H)
            c0 = c[:, :, :H]
            c2 = c[:, :, H:]
            c0s = jnp.concatenate(
                [jnp.zeros((1, B, H), f32), c0[:-1]], axis=0)
            h = jnp.maximum(
                a.reshape(rows // (2 * B), B, H) + c2 + c0s, 0.0)
            h = h.reshape(rows // 2, H)

        gi = jax.lax.dot_general(h, wih_refs[e][...], cdims,
                                 preferred_element_type=f32) + bih_refs[e][...]
        gi_scr[:, B * e:B * (e + 1), :] = gi.reshape(T, B, 3 * H)

    def step(t, hall):
        git = gi_scr[t]
        gh = jnp.concatenate(
            [jax.lax.dot_general(hall[B * e:B * (e + 1)], whh_refs[e][...],
                                 cdims, preferred_element_type=f32)
             + bhh_refs[e][...] for e in range(NE)], axis=0)
        r = jax.nn.sigmoid(git[:, :H] + gh[:, :H])
        z = jax.nn.sigmoid(git[:, H:2 * H] + gh[:, H:2 * H])
        n = jnp.tanh(git[:, 2 * H:] + r * gh[:, 2 * H:])
        return (1.0 - z) * n + z * hall

    hT = jax.lax.fori_loop(0, T, step, jnp.zeros((NB, H), f32), unroll=True)

    rg = jax.lax.broadcasted_iota(jnp.int32, (NB, NE * NC), 0) // B
    cg = jax.lax.broadcasted_iota(jnp.int32, (NB, NE * NC), 1)

    def vq_batch(z, ecat):
        d = (jnp.sum(z * z, axis=1, keepdims=True)
             - 2.0 * (z @ ecat.T)
             + jnp.sum(ecat * ecat, axis=1)[None, :])
        d = jnp.where(rg == (cg // NC), d, 1e30)
        dmin = jnp.min(d, axis=1, keepdims=True)
        idx = jnp.min(jnp.where(d == dmin, cg, NE * NC), axis=1)
        oh = (idx[:, None] == cg).astype(f32)
        zq = oh @ ecat
        seg = jnp.mean(oh.reshape(NE, B, NE * NC), axis=1)
        usage = -jnp.sum(seg * jnp.log(seg + 1e-10))
        loss = 1.4 * jnp.sum((zq - z) ** 2) / (B * H) + 0.01 * usage
        return z + (zq - z), loss

    ecat1 = jnp.concatenate([e1p[...], e1d[...], e1e[...]], axis=0)
    ecat2 = jnp.concatenate([e2p[...], e2d[...], e2e[...]], axis=0)
    q1, l1 = vq_batch(hT, ecat1)
    q2, l2 = vq_batch(hT - q1, ecat2)
    qa = jnp.concatenate([q1, q2], axis=1)
    for e in range(NE):
        q_scr[:, 2 * H * e:2 * H * (e + 1)] = qa[B * e:B * (e + 1)]

    x = q_scr[...]
    p = dp_ref[...]

    def lrelu(a):
        return jnp.where(a >= 0, a, 0.01 * a)

    def tap3(a, d, k):
        left = jnp.concatenate(
            [jnp.zeros((B, d), f32), a[:, :-d]], axis=1)
        right = jnp.concatenate(
            [a[:, d:], jnp.zeros((B, d), f32)], axis=1)
        return (p[0, k] * left + p[0, k + 1] * a
                + p[0, k + 2] * right + p[0, k + 3])

    hh = lrelu(tap3(x, 1, 0))
    hh = lrelu(tap3(hh, 3, 4))
    fp = tap3(hh, 5, 8) + x
    gap = jnp.mean(fp, axis=1, keepdims=True)
    gmp = jnp.max(fp, axis=1, keepdims=True)
    c1 = lrelu(p[0, 12] * gap + p[0, 13] * gmp)
    wc = jax.nn.sigmoid(p[0, 14] * c1)
    wt = jax.nn.sigmoid(p[0, 15])
    out_ref[...] = re_ref[...] + fp * (wc * wt)
    l_ref[0, :] = jnp.full((H,), l1 + l2, f32)


def _fold_conv(enc, i):
    w = enc['conv%d_w' % i]
    s = enc['bn%d_g' % i] / jnp.sqrt(enc['bn%d_v' % i] + 1e-5)
    bias = enc['bn%d_b' % i] - enc['bn%d_m' % i] * s
    ws = w * s[:, None, None]
    wt = jnp.transpose(ws, (2, 1, 0))
    return wt, bias[None, :]


def kernel(ref_embs, p_targets, d_targets, e_targets, params):
    f32 = jnp.float32
    encs = [params['enc_p'], params['enc_d'], params['enc_e']]


    wa_l, wc_l, bc_l = [], [], []
    w0_l, b0_l = [], []
    eyeb = jnp.eye(B, dtype=f32)[:, :, None]
    for enc in encs:
        w, b = _fold_conv(enc, 0)
        wp = jnp.pad(w[:, 0, :], ((0, 0), (0, H - CHANS[0])))
        w48 = (eyeb * wp[:, None, None, :]).reshape(3 * B, B * H)
        w0_l.append(w48)
        b0_l.append(jnp.tile(jnp.pad(b, ((0, 0), (0, H - CHANS[0]))),
                             (1, B)))
    w0 = jnp.stack(w0_l, 0)
    b0 = jnp.stack(b0_l, 0)
    for i in range(1, 6):
        ic, oc = CHANS[i - 1], CHANS[i]
        wa_e, wc_e, bc_e = [], [], []
        for enc in encs:
            w, b = _fold_conv(enc, i)
            wa_e.append(jnp.pad(w[1], ((0, H - ic), (0, H - oc))))
            wc_e.append(jnp.pad(
                jnp.concatenate(
                    [jnp.pad(w[0], ((0, 0), (0, H - oc))),
                     jnp.pad(w[2], ((0, 0), (0, H - oc)))], axis=1),
                ((0, H - ic), (0, 0))))
            bc_e.append(jnp.pad(b, ((0, 0), (0, H - oc))))
        wa_l.append(jnp.stack(wa_e, 0))
        wc_l.append(jnp.stack(wc_e, 0))
        bc_l.append(jnp.stack(bc_e, 0))
    wa = jnp.stack(wa_l, 0)
    wc = jnp.stack(wc_l, 0)
    bc = jnp.stack(bc_l, 0)

    gru_args = ([e['W_ih'] for e in encs] + [e['W_hh'] for e in encs]
                + [e['b_ih'].reshape(1, -1) for e in encs]
                + [e['b_hh'].reshape(1, -1) for e in encs])
    emb_args = [params[n] for n in ('vq_p_1', 'vq_d_1', 'vq_e_1',
                                    'vq_p_2', 'vq_d_2', 'vq_e_2')]

    da = params['da']
    dp = jnp.concatenate([
        da['rb_w1'][0, 0], da['rb_b1'],
        da['rb_w2'][0, 0], da['rb_b2'],
        da['rb_w3'][0, 0], da['rb_b3'],
        da['ca_w1'][0, :, 0], da['ca_w2'][0, 0], da['ta_b'],
    ]).reshape(1, 16).astype(f32)

    out, ltot = pl.pallas_call(
        _fused_kernel,
        out_shape=[
            jax.ShapeDtypeStruct((B, 6 * H), f32),
            jax.ShapeDtypeStruct((1, H), f32),
        ],
        scratch_shapes=[
            pltpu.VMEM((T, NB, 3 * H), f32),
            pltpu.VMEM((B, 6 * H), f32),
        ],
    )(p_targets, d_targets, e_targets, w0, b0, wa, wc, bc,
      *gru_args, *emb_args, ref_embs, dp)

    return out, ltot[0, 0]


# speedup vs baseline: 1.0218x; 1.0218x over previous
import jax
import jax.numpy as jnp
from jax.experimental import pallas as pl
from jax.experimental.pallas import tpu as pltpu

CHANS = (32, 32, 64, 64, 128, 128)
B = 16
L0 = 2048
T = 32
H = 128
NE = 3
NB = NE * B
NC = 7


def _fused_kernel(xp_ref, xd_ref, xe_ref, w0_ref, b0_ref, wa_ref, wc_ref, bc_ref,
                  wih0, wih1, wih2, whh0, whh1, whh2,
                  bih0, bih1, bih2, bhh0, bhh1, bhh2,
                  e1p, e1d, e1e, e2p, e2d, e2e, re_ref, dp_ref,
                  out_ref, l_ref, gi_scr, q_scr):
    f32 = jnp.float32
    x_ref = (xp_ref, xd_ref, xe_ref)
    wih_refs = (wih0, wih1, wih2)
    whh_refs = (whh0, whh1, whh2)
    bih_refs = (bih0, bih1, bih2)
    bhh_refs = (bhh0, bhh1, bhh2)
    cdims = (((1,), (1,)), ((), ()))
    eyeb = (jax.lax.broadcasted_iota(jnp.int32, (B, B), 0)
            == jax.lax.broadcasted_iota(jnp.int32, (B, B), 1)
            ).astype(f32)[:, :, None]

    for e in range(NE):
        x = x_ref[e][...].T
        xr = x.reshape(L0 // 2, 2, B)
        ev = xr[:, 0, :]
        od = xr[:, 1, :]
        pod = jnp.concatenate(
            [jnp.zeros((1, B), f32), od[:-1]], axis=0)
        x48 = jnp.concatenate([pod, ev, od], axis=1)
        wp = w0_ref[e]
        w48 = (eyeb * wp[:, None, None, :]).reshape(3 * B, B * H)
        b2048 = jnp.tile(b0_ref[e], (1, B))
        h = x48 @ w48 + b2048
        h = jnp.maximum(h, 0.0).reshape(L0 // 2 * B, H)

        for l in range(5):
            rows = h.shape[0]
            hr = h.reshape(rows // (2 * B), 2 * B, H)
            evf = hr[:, :B, :].reshape(rows // 2, H)
            odf = hr[:, B:, :].reshape(rows // 2, H)
            a = evf @ wa_ref[l, e] + bc_ref[l, e]
            c = (odf @ wc_ref[l, e]).reshape(rows // (2 * B), B, 2 * H)
            c0 = c[:, :, :H]
            c2 = c[:, :, H:]
            c0s = jnp.concatenate(
                [jnp.zeros((1, B, H), f32), c0[:-1]], axis=0)
            h = jnp.maximum(
                a.reshape(rows // (2 * B), B, H) + c2 + c0s, 0.0)
            h = h.reshape(rows // 2, H)

        gi = jax.lax.dot_general(h, wih_refs[e][...], cdims,
                                 preferred_element_type=f32) + bih_refs[e][...]
        gi_scr[:, B * e:B * (e + 1), :] = gi.reshape(T, B, 3 * H)

    def step(t, hall):
        git = gi_scr[t]
        gh = jnp.concatenate(
            [jax.lax.dot_general(hall[B * e:B * (e + 1)], whh_refs[e][...],
                                 cdims, preferred_element_type=f32)
             + bhh_refs[e][...] for e in range(NE)], axis=0)
        r = jax.nn.sigmoid(git[:, :H] + gh[:, :H])
        z = jax.nn.sigmoid(git[:, H:2 * H] + gh[:, H:2 * H])
        n = jnp.tanh(git[:, 2 * H:] + r * gh[:, 2 * H:])
        return (1.0 - z) * n + z * hall

    hT = jax.lax.fori_loop(0, T, step, jnp.zeros((NB, H), f32), unroll=True)

    rg = jax.lax.broadcasted_iota(jnp.int32, (NB, NE * NC), 0) // B
    cg = jax.lax.broadcasted_iota(jnp.int32, (NB, NE * NC), 1)

    def vq_batch(z, ecat):
        d = (jnp.sum(z * z, axis=1, keepdims=True)
             - 2.0 * (z @ ecat.T)
             + jnp.sum(ecat * ecat, axis=1)[None, :])
        d = jnp.where(rg == (cg // NC), d, 1e30)
        dmin = jnp.min(d, axis=1, keepdims=True)
        idx = jnp.min(jnp.where(d == dmin, cg, NE * NC), axis=1)
        oh = (idx[:, None] == cg).astype(f32)
        zq = oh @ ecat
        seg = jnp.mean(oh.reshape(NE, B, NE * NC), axis=1)
        usage = -jnp.sum(seg * jnp.log(seg + 1e-10))
        loss = 1.4 * jnp.sum((zq - z) ** 2) / (B * H) + 0.01 * usage
        return z + (zq - z), loss

    ecat1 = jnp.concatenate([e1p[...], e1d[...], e1e[...]], axis=0)
    ecat2 = jnp.concatenate([e2p[...], e2d[...], e2e[...]], axis=0)
    q1, l1 = vq_batch(hT, ecat1)
    q2, l2 = vq_batch(hT - q1, ecat2)
    qa = jnp.concatenate([q1, q2], axis=1)
    for e in range(NE):
        q_scr[:, 2 * H * e:2 * H * (e + 1)] = qa[B * e:B * (e + 1)]

    x = q_scr[...]
    p = dp_ref[...]

    def lrelu(a):
        return jnp.where(a >= 0, a, 0.01 * a)

    def tap3(a, d, k):
        left = jnp.concatenate(
            [jnp.zeros((B, d), f32), a[:, :-d]], axis=1)
        right = jnp.concatenate(
            [a[:, d:], jnp.zeros((B, d), f32)], axis=1)
        return (p[0, k] * left + p[0, k + 1] * a
                + p[0, k + 2] * right + p[0, k + 3])

    hh = lrelu(tap3(x, 1, 0))
    hh = lrelu(tap3(hh, 3, 4))
    fp = tap3(hh, 5, 8) + x
    gap = jnp.mean(fp, axis=1, keepdims=True)
    gmp = jnp.max(fp, axis=1, keepdims=True)
    c1 = lrelu(p[0, 12] * gap + p[0, 13] * gmp)
    wc = jax.nn.sigmoid(p[0, 14] * c1)
    wt = jax.nn.sigmoid(p[0, 15])
    out_ref[...] = re_ref[...] + fp * (wc * wt)
    l_ref[0, :] = jnp.full((H,), l1 + l2, f32)


def _fold_conv(enc, i):
    w = enc['conv%d_w' % i]
    s = enc['bn%d_g' % i] / jnp.sqrt(enc['bn%d_v' % i] + 1e-5)
    bias = enc['bn%d_b' % i] - enc['bn%d_m' % i] * s
    ws = w * s[:, None, None]
    wt = jnp.transpose(ws, (2, 1, 0))
    return wt, bias[None, :]


def kernel(ref_embs, p_targets, d_targets, e_targets, params):
    f32 = jnp.float32
    encs = [params['enc_p'], params['enc_d'], params['enc_e']]


    wa_l, wc_l, bc_l = [], [], []
    w0_l, b0_l = [], []
    for enc in encs:
        w, b = _fold_conv(enc, 0)
        w0_l.append(jnp.pad(w[:, 0, :], ((0, 0), (0, H - CHANS[0]))))
        b0_l.append(jnp.pad(b, ((0, 0), (0, H - CHANS[0]))))
    w0 = jnp.stack(w0_l, 0)
    b0 = jnp.stack(b0_l, 0)
    for i in range(1, 6):
        ic, oc = CHANS[i - 1], CHANS[i]
        wa_e, wc_e, bc_e = [], [], []
        for enc in encs:
            w, b = _fold_conv(enc, i)
            wa_e.append(jnp.pad(w[1], ((0, H - ic), (0, H - oc))))
            wc_e.append(jnp.pad(
                jnp.concatenate(
                    [jnp.pad(w[0], ((0, 0), (0, H - oc))),
                     jnp.pad(w[2], ((0, 0), (0, H - oc)))], axis=1),
                ((0, H - ic), (0, 0))))
            bc_e.append(jnp.pad(b, ((0, 0), (0, H - oc))))
        wa_l.append(jnp.stack(wa_e, 0))
        wc_l.append(jnp.stack(wc_e, 0))
        bc_l.append(jnp.stack(bc_e, 0))
    wa = jnp.stack(wa_l, 0)
    wc = jnp.stack(wc_l, 0)
    bc = jnp.stack(bc_l, 0)

    gru_args = ([e['W_ih'] for e in encs] + [e['W_hh'] for e in encs]
                + [e['b_ih'].reshape(1, -1) for e in encs]
                + [e['b_hh'].reshape(1, -1) for e in encs])
    emb_args = [params[n] for n in ('vq_p_1', 'vq_d_1', 'vq_e_1',
                                    'vq_p_2', 'vq_d_2', 'vq_e_2')]

    da = params['da']
    dp = jnp.concatenate([
        da['rb_w1'][0, 0], da['rb_b1'],
        da['rb_w2'][0, 0], da['rb_b2'],
        da['rb_w3'][0, 0], da['rb_b3'],
        da['ca_w1'][0, :, 0], da['ca_w2'][0, 0], da['ta_b'],
    ]).reshape(1, 16).astype(f32)

    out, ltot = pl.pallas_call(
        _fused_kernel,
        out_shape=[
            jax.ShapeDtypeStruct((B, 6 * H), f32),
            jax.ShapeDtypeStruct((1, H), f32),
        ],
        scratch_shapes=[
            pltpu.VMEM((T, NB, 3 * H), f32),
            pltpu.VMEM((B, 6 * H), f32),
        ],
    )(p_targets, d_targets, e_targets, w0, b0, wa, wc, bc,
      *gru_args, *emb_args, ref_embs, dp)

    return out, ltot[0, 0]
